# baseline (speedup 1.0000x reference)
import jax
import jax.numpy as jnp
from jax import lax
from jax.experimental import pallas as pl
from jax.experimental.pallas import tpu as pltpu
from jax.experimental.pallas import tpu_sc as plsc

_L = 16
_NW = 32


def _gather_body(ids_hbm, tok_hbm, out_hbm, idx_v, rows_v, sem, osem):
    b, seq_len = ids_hbm.shape
    h = tok_hbm.shape[1]
    t_per = (b * seq_len) // _NW
    n_grp = t_per // h
    chunks_per_seq = seq_len // t_per

    wid = lax.axis_index("s") * 2 + lax.axis_index("c")
    bi = wid // chunks_per_seq
    s0 = lax.rem(wid, chunks_per_seq) * t_per
    base = wid * t_per

    pltpu.sync_copy(ids_hbm.at[bi, pl.ds(s0, t_per)], idx_v)

    copies = []
    for k in range(n_grp):
        copies.append(pltpu.async_copy(
            tok_hbm.at[idx_v.at[pl.ds(k * h, h)]],
            rows_v.at[pl.ds(k * h, h)], sem))
    out_cps = []
    for k in range(n_grp):
        copies[k].wait()
        out_cps.append(pltpu.async_copy(
            rows_v.at[pl.ds(k * h, h)],
            out_hbm.at[pl.ds(base + k * h, h)], osem))
    for c in out_cps:
        c.wait()


def _ln_body(tok_ref, ps_ref, gam_ref, bet_ref, out_ref):
    e = tok_ref[...] + ps_ref[...]
    mean = jnp.mean(e, axis=-1, keepdims=True)
    c = e - mean
    var = jnp.mean(c * c, axis=-1, keepdims=True)
    r = lax.rsqrt(var + jnp.float32(1e-5))
    out_ref[...] = c * r * gam_ref[...] + bet_ref[...]


def kernel(input_ids, token_type_ids, tok_table, pos_table, seg_table,
           gamma, beta):
    b, s = input_ids.shape
    v, h = tok_table.shape
    n = b * s
    t_per = n // _NW
    ids = input_ids.astype(jnp.int32)

    mesh = plsc.VectorSubcoreMesh(core_axis_name="c", subcore_axis_name="s")
    tok_rows = pl.kernel(
        _gather_body,
        out_type=jax.ShapeDtypeStruct((n, h), jnp.float32),
        mesh=mesh,
        scratch_types=[
            pltpu.VMEM((t_per,), jnp.int32),
            pltpu.VMEM((t_per, h), jnp.float32),
            pltpu.SemaphoreType.DMA,
            pltpu.SemaphoreType.DMA,
        ],
    )(ids, tok_table)

    posseg = (pos_table[None, :, :]
              + jnp.where((token_type_ids == 0)[..., None],
                          seg_table[0], seg_table[1])).reshape(n, h)

    br = 8192
    out = pl.pallas_call(
        _ln_body,
        grid=(n // br,),
        in_specs=[
            pl.BlockSpec((br, h), lambda i: (i, 0)),
            pl.BlockSpec((br, h), lambda i: (i, 0)),
            pl.BlockSpec((1, h), lambda i: (0, 0)),
            pl.BlockSpec((1, h), lambda i: (0, 0)),
        ],
        out_specs=pl.BlockSpec((br, h), lambda i: (i, 0)),
        out_shape=jax.ShapeDtypeStruct((n, h), jnp.float32),
    )(tok_rows, posseg, gamma.reshape(1, h), beta.reshape(1, h))
    return out.reshape(b, s, h)


# speedup vs baseline: 1.0662x; 1.0662x over previous
import jax
import jax.numpy as jnp
from jax import lax
from jax.experimental import pallas as pl
from jax.experimental.pallas import tpu as pltpu
from jax.experimental.pallas import tpu_sc as plsc

_L = 16
_NW = 32


def _gather_body(ids_hbm, tok_hbm, out_hbm, idx_v, rows_v, sem, osem):
    b, seq_len = ids_hbm.shape
    h = tok_hbm.shape[1]
    t_per = (b * seq_len) // _NW
    n_grp = t_per // h
    chunks_per_seq = seq_len // t_per

    wid = lax.axis_index("s") * 2 + lax.axis_index("c")
    bi = wid // chunks_per_seq
    s0 = lax.rem(wid, chunks_per_seq) * t_per
    base = wid * t_per

    pltpu.sync_copy(ids_hbm.at[bi, pl.ds(s0, t_per)], idx_v)

    copies = []
    for k in range(n_grp):
        copies.append(pltpu.async_copy(
            tok_hbm.at[idx_v.at[pl.ds(k * h, h)]],
            rows_v.at[pl.ds(k * h, h)], sem))
    out_cps = []
    for k in range(n_grp):
        copies[k].wait()
        out_cps.append(pltpu.async_copy(
            rows_v.at[pl.ds(k * h, h)],
            out_hbm.at[pl.ds(base + k * h, h)], osem))
    for c in out_cps:
        c.wait()


def _ln_body(tok_ref, ps_ref, gam_ref, bet_ref, out_ref):
    e = tok_ref[...] + ps_ref[...]
    mean = jnp.mean(e, axis=-1, keepdims=True)
    c = e - mean
    var = jnp.mean(c * c, axis=-1, keepdims=True)
    r = lax.rsqrt(var + jnp.float32(1e-5))
    out_ref[...] = c * r * gam_ref[...] + bet_ref[...]


def kernel(input_ids, token_type_ids, tok_table, pos_table, seg_table,
           gamma, beta):
    b, s = input_ids.shape
    v, h = tok_table.shape
    n = b * s
    t_per = n // _NW
    ids = input_ids.astype(jnp.int32)

    mesh = plsc.VectorSubcoreMesh(core_axis_name="c", subcore_axis_name="s")
    tok_rows = pl.kernel(
        _gather_body,
        out_type=jax.ShapeDtypeStruct((n, h), jnp.float32),
        mesh=mesh,
        scratch_types=[
            pltpu.VMEM((t_per,), jnp.int32),
            pltpu.VMEM((t_per, h), jnp.float32),
            pltpu.SemaphoreType.DMA,
            pltpu.SemaphoreType.DMA,
        ],
    )(ids, tok_table)

    posseg = (pos_table[None, :, :]
              + jnp.where((token_type_ids == 0)[..., None],
                          seg_table[0], seg_table[1])).reshape(n, h)

    br = 4096
    out = pl.pallas_call(
        _ln_body,
        grid=(n // br,),
        in_specs=[
            pl.BlockSpec((br, h), lambda i: (i, 0)),
            pl.BlockSpec((br, h), lambda i: (i, 0)),
            pl.BlockSpec((1, h), lambda i: (0, 0)),
            pl.BlockSpec((1, h), lambda i: (0, 0)),
        ],
        out_specs=pl.BlockSpec((br, h), lambda i: (i, 0)),
        out_shape=jax.ShapeDtypeStruct((n, h), jnp.float32),
    )(tok_rows, posseg, gamma.reshape(1, h), beta.reshape(1, h))
    return out.reshape(b, s, h)
